# Initial kernel scaffold; baseline (speedup 1.0000x reference)
import jax
import jax.numpy as jnp
from jax.experimental import pallas as pl


def kernel(pred_boxes, gt_boxes):
    raise NotImplementedError("write your pallas kernel here")


# speedup vs baseline: 7.0354x; 7.0354x over previous
import jax
import jax.numpy as jnp
from jax import lax
from jax.experimental import pallas as pl
from jax.experimental.pallas import tpu as pltpu

K = 64
NS = 9
G = 64
LANES = 128


def _matcher_body(predT_ref, gt_ref, out_ref):
    npad = predT_ref.shape[-1]

    predT = predT_ref[0]
    gtb = gt_ref[0]

    pcx = predT[0:1, :]
    pcy = predT[1:2, :]
    pw = predT[2:3, :]
    ph = predT[3:4, :]

    gcx = gtb[:, 0:1]
    gcy = gtb[:, 1:2]
    gw = gtb[:, 2:3]
    gh = gtb[:, 3:4]

    d0 = (gcx - pcx) * (gcx - pcx)
    d1 = (gcy - pcy) * (gcy - pcy)
    d2c = (gw - pw) * (gw - pw)
    d3 = (gh - ph) * (gh - ph)
    dist = jnp.sqrt(((d0 + d1) + d2c) + d3 + 1e-12)

    px0 = pcx - 0.5 * pw
    py0 = pcy - 0.5 * ph
    px1 = pcx + 0.5 * pw
    py1 = pcy + 0.5 * ph
    gx0 = gcx - 0.5 * gw
    gy0 = gcy - 0.5 * gh
    gx1 = gcx + 0.5 * gw
    gy1 = gcy + 0.5 * gh

    ltx = jnp.maximum(gx0, px0)
    lty = jnp.maximum(gy0, py0)
    rbx = jnp.minimum(gx1, px1)
    rby = jnp.minimum(gy1, py1)
    iw = jnp.clip(rbx - ltx, 0.0)
    ih = jnp.clip(rby - lty, 0.0)
    inter = iw * ih
    area_g = (gx1 - gx0) * (gy1 - gy0)
    area_k = (px1 - px0) * (py1 - py0)
    union = area_g + area_k - inter
    iou = jnp.where(union > 0, inter / jnp.where(union > 0, union, 1.0), 0.0)

    iota_n = lax.broadcasted_iota(jnp.int32, (G, npad), 1)
    colk = lax.broadcasted_iota(jnp.int32, (G, K), 1)
    col9 = lax.broadcasted_iota(jnp.int32, (G, NS), 1)

    def run_scoped_body(dscr):
        dscr[...] = dist

        def body(i, carry):
            kidx, kiou = carry
            d = dscr[...]
            m = jnp.min(d, axis=1, keepdims=True)
            sel_key = jnp.where(d == m, iota_n, npad)
            idx = jnp.min(sel_key, axis=1, keepdims=True)
            onehot = iota_n == idx
            iou_i = jnp.max(jnp.where(onehot, iou, -1.0), axis=1, keepdims=True)
            dscr[...] = jnp.where(onehot, jnp.float32(jnp.inf), d)
            kidx = jnp.where(colk == i, idx, kidx)
            kiou = jnp.where(colk == i, iou_i, kiou)
            return kidx, kiou

        kidx0 = jnp.zeros((G, K), jnp.int32)
        kiou0 = jnp.zeros((G, K), jnp.float32)
        kidx, kiou = lax.fori_loop(0, K, body, (kidx0, kiou0))

        def body9(j, carry):
            kiou_c, outsel = carry
            m = jnp.max(kiou_c, axis=1, keepdims=True)
            pos = jnp.min(jnp.where(kiou_c == m, colk, K), axis=1, keepdims=True)
            hit = colk == pos
            pidx = jnp.sum(jnp.where(hit, kidx, 0), axis=1, keepdims=True)
            kiou_c = jnp.where(hit, -jnp.float32(jnp.inf), kiou_c)
            outsel = jnp.where(col9 == j, pidx, outsel)
            return kiou_c, outsel

        out0 = jnp.zeros((G, NS), jnp.int32)
        _, outsel = lax.fori_loop(0, NS, body9, (kiou, out0))
        out_ref[0] = outsel

    pl.run_scoped(run_scoped_body, pltpu.VMEM((G, dist.shape[-1]), jnp.float32))


def kernel(pred_boxes, gt_boxes):
    B, N, _ = pred_boxes.shape
    npad = ((N + LANES - 1) // LANES) * LANES
    pred_pad = jnp.pad(pred_boxes, ((0, 0), (0, npad - N), (0, 0)),
                       constant_values=4.0)
    predT = pred_pad.transpose(0, 2, 1)

    out = pl.pallas_call(
        _matcher_body,
        grid=(B,),
        in_specs=[
            pl.BlockSpec((1, 4, npad), lambda b: (b, 0, 0)),
            pl.BlockSpec((1, G, 4), lambda b: (b, 0, 0)),
        ],
        out_specs=pl.BlockSpec((1, G, NS), lambda b: (b, 0, 0)),
        out_shape=jax.ShapeDtypeStruct((B, G, NS), jnp.int32),
    )(predT, gt_boxes)

    pred_idx = out.reshape(B, G * NS)
    gt_idx = jnp.broadcast_to(
        jnp.arange(G, dtype=jnp.int32)[None, :, None], (B, G, NS)
    ).reshape(B, G * NS)
    return pred_idx, gt_idx
